# baseline (speedup 1.0000x reference)
import dataclasses
import functools

import jax
import jax.numpy as jnp
from jax import lax
from jax.experimental import pallas as pl
from jax.experimental.pallas import tpu as pltpu
from jax.experimental.pallas import tpu_sc as plsc

TOKENS = 16384
D = 1024
K = 8
L = 16
NW = 32
TPW = TOKENS // NW
G = 4
GK = G * K
NG = TPW // G
NC = D // L
NB = 2
UNROLL = 4


def _sc_kernel(x_hbm, idx_hbm, w_hbm, tbl_hbm, out_hbm,
               idx_v, w_v, rows_v, x_v, o_v, sem_r, sem_x):
    wid = lax.axis_index("s") * 2 + lax.axis_index("c")
    t0 = wid * TPW

    pltpu.sync_copy(idx_hbm.at[pl.ds(t0 * K, TPW * K)], idx_v)
    pltpu.sync_copy(w_hbm.at[pl.ds(t0 * K, TPW * K)], w_v)

    def in_copies(g, b):
        return (
            pltpu.make_async_copy(
                x_hbm.at[pl.ds(t0 + g * G, G)], x_v.at[b], sem_x.at[b]),
            pltpu.make_async_copy(
                tbl_hbm.at[idx_v.at[pl.ds(g * GK, GK)]], rows_v.at[b],
                sem_r.at[b]),
        )

    def start_in(g, b):
        for c in in_copies(g, b):
            c.start()

    def wait_in(g, b):
        for c in in_copies(g, b):
            c.wait()

    start_in(0, 0)

    @pl.loop(0, NG, step=NB)
    def _group(g0):
        for b in range(NB):
            g = g0 + b
            nb = (b + 1) % NB

            @pl.when(g + 1 < NG)
            def _():
                start_in(g + 1, nb)

            wait_in(g, b)

            wvecs = [w_v[pl.ds(g * GK + j * L, L)] for j in range(GK // L)]

            for i in range(G):
                def dot_body(c, accs, _i=i, _b=b):
                    xc = x_v[_b, _i, pl.ds(c * L, L)]
                    return tuple(
                        accs[k] + xc * rows_v[_b, _i * K + k, pl.ds(c * L, L)]
                        for k in range(K)
                    )

                accs = lax.fori_loop(
                    0, NC, dot_body,
                    tuple(jnp.zeros((L,), jnp.float32) for _ in range(K)),
                    unroll=UNROLL,
                )

                weff = []
                for k in range(K):
                    p = jnp.sum(accs[k])
                    pv = jnp.full((L,), p, jnp.float32)
                    e = jnp.exp(-2.0 * jnp.abs(pv))
                    th = jnp.sign(pv) * (1.0 - e) / (1.0 + e)
                    j = i * K + k
                    weff.append(th * wvecs[j // L][j % L])

                def comb_body(c, carry, _i=i, _b=b, _weff=weff):
                    s = pl.ds(c * L, L)
                    acc = x_v[_b, _i, s]
                    for k in range(K):
                        acc = acc + _weff[k] * rows_v[_b, _i * K + k, s]
                    o_v[_i, s] = acc
                    return carry

                lax.fori_loop(0, NC, comb_body, 0, unroll=UNROLL)

            pltpu.sync_copy(o_v, out_hbm.at[pl.ds(t0 + g * G, G)])


def kernel(x, indices, weights, table):
    idx_flat = indices.astype(jnp.int32).reshape(-1)
    w_flat = weights.reshape(-1)
    mesh = plsc.VectorSubcoreMesh(core_axis_name="c", subcore_axis_name="s")
    cp = pltpu.CompilerParams()
    if "needs_layout_passes" in pltpu.CompilerParams.__dataclass_fields__:
        cp = dataclasses.replace(cp, needs_layout_passes=False)
    f = pl.kernel(
        _sc_kernel,
        mesh=mesh,
        compiler_params=cp,
        out_type=jax.ShapeDtypeStruct((TOKENS, D), jnp.float32),
        scratch_types=[
            pltpu.VMEM((TPW * K,), jnp.int32),
            pltpu.VMEM((TPW * K,), jnp.float32),
            pltpu.VMEM((NB, GK, D), jnp.float32),
            pltpu.VMEM((NB, G, D), jnp.float32),
            pltpu.VMEM((G, D), jnp.float32),
            pltpu.SemaphoreType.DMA((NB,)),
            pltpu.SemaphoreType.DMA((NB,)),
        ],
    )
    return f(x, idx_flat, w_flat, table)


# speedup vs baseline: 1.0828x; 1.0828x over previous
import dataclasses
import functools

import jax
import jax.numpy as jnp
from jax import lax
from jax.experimental import pallas as pl
from jax.experimental.pallas import tpu as pltpu
from jax.experimental.pallas import tpu_sc as plsc

TOKENS = 16384
D = 1024
K = 8
L = 16
NW = 32
TPW = TOKENS // NW
G = 4
GK = G * K
NG = TPW // G
NC = D // L
NB = 2
UNROLL = 1


def _sc_kernel(x_hbm, idx_hbm, w_hbm, tbl_hbm, out_hbm,
               idx_v, w_v, rows_v, x_v, o_v, sem_r, sem_x):
    wid = lax.axis_index("s") * 2 + lax.axis_index("c")
    t0 = wid * TPW

    pltpu.sync_copy(idx_hbm.at[pl.ds(t0 * K, TPW * K)], idx_v)
    pltpu.sync_copy(w_hbm.at[pl.ds(t0 * K, TPW * K)], w_v)

    def in_copies(g, b):
        return (
            pltpu.make_async_copy(
                x_hbm.at[pl.ds(t0 + g * G, G)], x_v.at[b], sem_x.at[b]),
            pltpu.make_async_copy(
                tbl_hbm.at[idx_v.at[pl.ds(g * GK, GK)]], rows_v.at[b],
                sem_r.at[b]),
        )

    def start_in(g, b):
        for c in in_copies(g, b):
            c.start()

    def wait_in(g, b):
        for c in in_copies(g, b):
            c.wait()

    start_in(0, 0)

    @pl.loop(0, NG, step=NB)
    def _group(g0):
        for b in range(NB):
            g = g0 + b
            nb = (b + 1) % NB

            @pl.when(g + 1 < NG)
            def _():
                start_in(g + 1, nb)

            wait_in(g, b)

            wvecs = [w_v[pl.ds(g * GK + j * L, L)] for j in range(GK // L)]

            for i in range(G):
                def dot_body(c, accs, _i=i, _b=b):
                    xc = x_v[_b, _i, pl.ds(c * L, L)]
                    return tuple(
                        accs[k] + xc * rows_v[_b, _i * K + k, pl.ds(c * L, L)]
                        for k in range(K)
                    )

                accs = lax.fori_loop(
                    0, NC, dot_body,
                    tuple(jnp.zeros((L,), jnp.float32) for _ in range(K)),
                    unroll=UNROLL,
                )

                weff = []
                for k in range(K):
                    p = jnp.sum(accs[k])
                    pv = jnp.full((L,), p, jnp.float32)
                    e = jnp.exp(-2.0 * jnp.abs(pv))
                    th = jnp.sign(pv) * (1.0 - e) / (1.0 + e)
                    j = i * K + k
                    weff.append(th * wvecs[j // L][j % L])

                def comb_body(c, carry, _i=i, _b=b, _weff=weff):
                    s = pl.ds(c * L, L)
                    acc = x_v[_b, _i, s]
                    for k in range(K):
                        acc = acc + _weff[k] * rows_v[_b, _i * K + k, s]
                    o_v[_i, s] = acc
                    return carry

                lax.fori_loop(0, NC, comb_body, 0, unroll=UNROLL)

            pltpu.sync_copy(o_v, out_hbm.at[pl.ds(t0 + g * G, G)])


def kernel(x, indices, weights, table):
    idx_flat = indices.astype(jnp.int32).reshape(-1)
    w_flat = weights.reshape(-1)
    mesh = plsc.VectorSubcoreMesh(core_axis_name="c", subcore_axis_name="s")
    cp = pltpu.CompilerParams()
    if "needs_layout_passes" in pltpu.CompilerParams.__dataclass_fields__:
        cp = dataclasses.replace(cp, needs_layout_passes=False)
    f = pl.kernel(
        _sc_kernel,
        mesh=mesh,
        compiler_params=cp,
        out_type=jax.ShapeDtypeStruct((TOKENS, D), jnp.float32),
        scratch_types=[
            pltpu.VMEM((TPW * K,), jnp.int32),
            pltpu.VMEM((TPW * K,), jnp.float32),
            pltpu.VMEM((NB, GK, D), jnp.float32),
            pltpu.VMEM((NB, G, D), jnp.float32),
            pltpu.VMEM((G, D), jnp.float32),
            pltpu.SemaphoreType.DMA((NB,)),
            pltpu.SemaphoreType.DMA((NB,)),
        ],
    )
    return f(x, idx_flat, w_flat, table)


# speedup vs baseline: 1.1767x; 1.0868x over previous
import dataclasses
import functools

import jax
import jax.numpy as jnp
from jax import lax
from jax.experimental import pallas as pl
from jax.experimental.pallas import tpu as pltpu
from jax.experimental.pallas import tpu_sc as plsc

TOKENS = 16384
D = 1024
K = 8
L = 16
NW = 32
TPW = TOKENS // NW
G = 4
GK = G * K
NG = TPW // G
NC = D // L
NB = 2
UNROLL = 1


def _sc_kernel(x_hbm, idx_hbm, w_hbm, tbl_hbm, out_hbm,
               idx_v, w_v, rows_v, x_v, o_v, sem_r, sem_x):
    wid = lax.axis_index("s") * 2 + lax.axis_index("c")
    t0 = wid * TPW

    pltpu.sync_copy(idx_hbm.at[pl.ds(t0 * K, TPW * K)], idx_v)
    pltpu.sync_copy(w_hbm.at[pl.ds(t0 * K, TPW * K)], w_v)

    def in_copies(g, b):
        return (
            pltpu.make_async_copy(
                x_hbm.at[pl.ds(t0 + g * G, G)], x_v.at[b], sem_x.at[b]),
            pltpu.make_async_copy(
                tbl_hbm.at[idx_v.at[pl.ds(g * GK, GK)]], rows_v.at[b],
                sem_r.at[b]),
        )

    def start_in(g, b):
        for c in in_copies(g, b):
            c.start()

    def wait_in(g, b):
        for c in in_copies(g, b):
            c.wait()

    start_in(0, 0)

    @pl.loop(0, NG, step=NB)
    def _group(g0):
        for b in range(NB):
            g = g0 + b
            nb = (b + 1) % NB

            @pl.when(g + 1 < NG)
            def _():
                start_in(g + 1, nb)

            wait_in(g, b)

            wvecs = [w_v[pl.ds(g * GK + j * L, L)] for j in range(GK // L)]

            for i in range(G):
                def dot_body(c, accs, _i=i, _b=b):
                    xc = x_v[_b, _i, pl.ds(c * L, L)]
                    return tuple(
                        accs[k] + xc * rows_v[_b, _i * K + k, pl.ds(c * L, L)]
                        for k in range(K)
                    )

                accs = lax.fori_loop(
                    0, NC, dot_body,
                    tuple(jnp.zeros((L,), jnp.float32) for _ in range(K)),
                    unroll=UNROLL,
                )

                weff = []
                for k in range(K):
                    p = jnp.sum(accs[k])
                    pv = jnp.full((L,), p, jnp.float32)
                    e = jnp.exp(-2.0 * jnp.abs(pv))
                    th = jnp.sign(pv) * (1.0 - e) / (1.0 + e)
                    j = i * K + k
                    weff.append(th * wvecs[j // L][j % L])

                def comb_body(c, carry, _i=i, _b=b, _weff=weff):
                    s = pl.ds(c * L, L)
                    m = [_weff[k] * rows_v[_b, _i * K + k, s] for k in range(K)]
                    t0, t1 = m[0] + m[1], m[2] + m[3]
                    t2, t3 = m[4] + m[5], m[6] + m[7]
                    o_v[_i, s] = (x_v[_b, _i, s] + (t0 + t1)) + (t2 + t3)
                    return carry

                lax.fori_loop(0, NC, comb_body, 0, unroll=UNROLL)

            pltpu.sync_copy(o_v, out_hbm.at[pl.ds(t0 + g * G, G)])


def kernel(x, indices, weights, table):
    idx_flat = indices.astype(jnp.int32).reshape(-1)
    w_flat = weights.reshape(-1)
    mesh = plsc.VectorSubcoreMesh(core_axis_name="c", subcore_axis_name="s")
    cp = pltpu.CompilerParams()
    if "needs_layout_passes" in pltpu.CompilerParams.__dataclass_fields__:
        cp = dataclasses.replace(cp, needs_layout_passes=False)
    f = pl.kernel(
        _sc_kernel,
        mesh=mesh,
        compiler_params=cp,
        out_type=jax.ShapeDtypeStruct((TOKENS, D), jnp.float32),
        scratch_types=[
            pltpu.VMEM((TPW * K,), jnp.int32),
            pltpu.VMEM((TPW * K,), jnp.float32),
            pltpu.VMEM((NB, GK, D), jnp.float32),
            pltpu.VMEM((NB, G, D), jnp.float32),
            pltpu.VMEM((G, D), jnp.float32),
            pltpu.SemaphoreType.DMA((NB,)),
            pltpu.SemaphoreType.DMA((NB,)),
        ],
    )
    return f(x, idx_flat, w_flat, table)
